# Initial kernel scaffold; baseline (speedup 1.0000x reference)
import jax
import jax.numpy as jnp
from jax.experimental import pallas as pl


def kernel(x, conv1_w, conv1_b, bbox_w, bbox_b, cls_w, cls_b, ang_w, ang_b):
    raise NotImplementedError("write your pallas kernel here")


# speedup vs baseline: 6.5635x; 6.5635x over previous
import functools

import jax
import jax.numpy as jnp
from jax import lax
from jax.experimental import pallas as pl
from jax.experimental.pallas import tpu as pltpu

_NUM_CLASSES = 2
_MAX_DET = 300
_SCORE_THR = 0.001
_IOU_THR = 0.7
_NL = 640
_HP = jax.lax.Precision.HIGHEST

_rp = functools.partial(jax.lax.reduce_precision, exponent_bits=8, mantissa_bits=7)


def _backbone_kernel(x_ref, w_ref, b_ref, out_ref, xp_ref):
    co = pl.program_id(1)

    @pl.when(co == 0)
    def _pad():
        xp_ref[...] = jnp.zeros_like(xp_ref)
        xp_ref[:, 1:225, 1:225] = x_ref[0]

    riota = lax.broadcasted_iota(jnp.int32, (224, 224), 0)
    ciota = lax.broadcasted_iota(jnp.int32, (224, 224), 1)
    even = ((riota % 2) == 0) & ((ciota % 2) == 0)

    acc = jnp.zeros((224, 224), jnp.float32)
    for ci in range(3):
        for ky in range(3):
            for kx in range(3):
                w = w_ref[0, co, ci * 9 + ky * 3 + kx]
                acc = acc + w * xp_ref[ci, ky:ky + 224, kx:kx + 224]
    a = jnp.maximum(acc + b_ref[0, co], 0.0)
    rshift = jnp.concatenate([a[1:, :], a[:1, :]], axis=0)
    m1 = jnp.maximum(a, rshift)
    cshift = jnp.concatenate([m1[:, 1:], m1[:, :1]], axis=1)
    m2 = jnp.maximum(m1, cshift)
    pooled = jnp.where(even, m2, 0.0)
    pooled = pooled.astype(jnp.bfloat16).astype(jnp.float32)
    val = jnp.full((1, 1, 128), jnp.sum(pooled) / 12544.0, jnp.float32)
    out_ref[:, pl.ds(co, 1), :] = val


def _backbone_call(xb, wb, bias):
    out = pl.pallas_call(
        _backbone_kernel,
        grid=(4, 16),
        in_specs=[
            pl.BlockSpec((1, 3, 224, 224), lambda b, c: (b, 0, 0, 0)),
            pl.BlockSpec((1, 16, 32), lambda b, c: (0, 0, 0)),
            pl.BlockSpec((1, 16), lambda b, c: (0, 0), memory_space=pltpu.SMEM),
        ],
        out_specs=pl.BlockSpec((1, 16, 128), lambda b, c: (b, 0, 0)),
        out_shape=jax.ShapeDtypeStruct((4, 16, 128), jnp.float32),
        scratch_shapes=[pltpu.VMEM((3, 226, 226), jnp.float32)],
    )(xb, wb, bias)
    return out[:, :, 0]


def _heads_kernel(fm_ref, wb_ref, wc_ref, wa_ref, bb_ref, bc_ref, ba_ref,
                  ob_ref, oc_ref, oa_ref):
    fm = fm_ref[...]

    def head(w_ref, b_ref, o_ref, n):
        acc = jnp.zeros((4, n), jnp.float32)
        for c in range(16):
            acc = acc + fm[:, c:c + 1] * w_ref[c:c + 1, :]
        o_ref[...] = acc + b_ref[...]

    head(wb_ref, bb_ref, ob_ref, 1200)
    head(wc_ref, bc_ref, oc_ref, 600)
    head(wa_ref, ba_ref, oa_ref, 300)


def _heads_call(fmean, wbT, wcT, waT, bb, bc, ba):
    return pl.pallas_call(
        _heads_kernel,
        out_shape=(
            jax.ShapeDtypeStruct((4, 1200), jnp.float32),
            jax.ShapeDtypeStruct((4, 600), jnp.float32),
            jax.ShapeDtypeStruct((4, 300), jnp.float32),
        ),
    )(fmean, wbT, wcT, waT, bb, bc, ba)


def _nms_kernel(dc_ref, dt_ref, out_ref, iou_ref):
    f32 = jnp.float32
    dc = dc_ref[0]
    dt = dt_ref[0]
    li = lax.broadcasted_iota(jnp.int32, (_NL, _NL), 1)
    si = lax.broadcasted_iota(jnp.int32, (_NL, _NL), 0)

    s_row = dt[5:6, :]
    s_col = dc[:, 5:6]
    sr = jnp.broadcast_to(s_row, (_NL, _NL))
    sc = jnp.broadcast_to(s_col, (_NL, _NL))
    beats_ji = (sr > sc) | ((sr == sc) & (li < si))
    rank_col = jnp.sum(beats_ji.astype(f32), axis=1, keepdims=True)
    beats_sj = (sc > sr) | ((sc == sr) & (si < li))
    rank_row = jnp.sum(beats_sj.astype(f32), axis=0, keepdims=True)

    P = (jnp.broadcast_to(rank_row, (_NL, _NL)) == si.astype(f32)).astype(f32)
    sorted_c = jnp.dot(P, dc, precision=_HP, preferred_element_type=f32)
    PT = (jnp.broadcast_to(rank_col, (_NL, _NL)) == li.astype(f32)).astype(f32)
    sorted_t = jnp.dot(dt, PT, precision=_HP, preferred_element_type=f32)

    eps = 1e-7

    def covars(xv, yv, wv, hv, rv):
        a = wv * wv / 12.0
        bb = hv * hv / 12.0
        cth = jnp.cos(rv)
        sth = jnp.sin(rv)
        A = a * cth * cth + bb * sth * sth
        Bv = a * sth * sth + bb * cth * cth
        Cv = (a - bb) * cth * sth
        det = jnp.clip(A * Bv - Cv * Cv, 0.0, None)
        return A, Bv, Cv, det

    lab_r = sorted_t[6:7, :]
    off_r = lab_r * 1.0e4
    x_r = sorted_t[0:1, :] + off_r
    y_r = sorted_t[1:2, :] + off_r
    A_r, B_r, C_r, det_r = covars(x_r, y_r, sorted_t[2:3, :], sorted_t[3:4, :],
                                  sorted_t[4:5, :])
    lab_c = sorted_c[:, 6:7]
    off_c = lab_c * 1.0e4
    x_c = sorted_c[:, 0:1] + off_c
    y_c = sorted_c[:, 1:2] + off_c
    A_c, B_c, C_c, det_c = covars(x_c, y_c, sorted_c[:, 2:3], sorted_c[:, 3:4],
                                  sorted_c[:, 4:5])

    bc2 = lambda v: jnp.broadcast_to(v, (_NL, _NL))
    dx = bc2(x_c) - bc2(x_r)
    dy = bc2(y_c) - bc2(y_r)
    Va = bc2(A_c) + bc2(A_r)
    Vb = bc2(B_c) + bc2(B_r)
    Vc = bc2(C_c) + bc2(C_r)
    denom = Va * Vb - Vc * Vc + eps
    t1 = (Va * dy * dy + Vb * dx * dx) / denom * 0.25
    t2 = -(Vc * dx * dy) / denom * 0.5
    t3 = 0.5 * jnp.log(denom / (4.0 * jnp.sqrt(bc2(det_c) * bc2(det_r)) + eps) + eps)
    bd = jnp.clip(t1 + t2 + t3, eps, 100.0)
    hd = jnp.sqrt(1.0 - jnp.exp(-bd) + eps)
    iou_ref[...] = 1.0 - hd

    s_sorted = sorted_t[5:6, :]
    keep0 = (s_sorted > _SCORE_THR).astype(f32)
    lane1 = lax.broadcasted_iota(jnp.int32, (1, _NL), 1)

    def body(i, keep):
        row = iou_ref[pl.ds(i, 1), :]
        k_i = jnp.sum(keep * (lane1 == i).astype(f32))
        sup = (row > _IOU_THR).astype(f32) * (lane1 > i).astype(f32) * k_i
        return keep * (1.0 - sup)

    keep = lax.fori_loop(0, 600, body, keep0)

    LT = (si <= li).astype(f32)
    kpos = jnp.dot(keep, LT, precision=_HP, preferred_element_type=f32)
    r320 = lax.broadcasted_iota(jnp.int32, (320, _NL), 0).astype(f32)
    Q = ((jnp.broadcast_to(keep, (320, _NL)) > 0.0)
         & (jnp.broadcast_to(kpos, (320, _NL)) == r320 + 1.0)).astype(f32)
    out_ref[0] = jnp.dot(Q, sorted_c, precision=_HP, preferred_element_type=f32)


def _nms_call(dets_c, dets_t):
    return pl.pallas_call(
        _nms_kernel,
        grid=(4,),
        in_specs=[
            pl.BlockSpec((1, _NL, 8), lambda b: (b, 0, 0)),
            pl.BlockSpec((1, 8, _NL), lambda b: (b, 0, 0)),
        ],
        out_specs=pl.BlockSpec((1, 320, 8), lambda b: (b, 0, 0)),
        out_shape=jax.ShapeDtypeStruct((4, 320, 8), jnp.float32),
        scratch_shapes=[pltpu.VMEM((_NL, _NL), jnp.float32)],
    )(dets_c, dets_t)


def kernel(x, conv1_w, conv1_b, bbox_w, bbox_b, cls_w, cls_b, ang_w, ang_b):
    B = x.shape[0]
    xb = _rp(x)
    wb = _rp(conv1_w).reshape(1, 16, 27)
    wb = jnp.concatenate([wb, jnp.zeros((1, 16, 5), jnp.float32)], axis=-1)
    fmean = _backbone_call(xb, wb, conv1_b.reshape(1, 16))

    wbT = _rp(bbox_w.reshape(1200, 16)).T
    wcT = _rp(cls_w.reshape(600, 16)).T
    waT = _rp(ang_w.reshape(300, 16)).T
    boxes_l, cls_l, ang_l = _heads_call(
        fmean, wbT, wcT, waT,
        bbox_b.reshape(1, 1200), cls_b.reshape(1, 600), ang_b.reshape(1, 300))
    boxes = boxes_l.reshape(B, _MAX_DET, 4)
    slog = cls_l.reshape(B, _MAX_DET, _NUM_CLASSES)
    scores = jax.nn.softmax(slog, axis=2)
    angles = ang_l.reshape(B, _MAX_DET, 1)

    boxes5 = jnp.concatenate([boxes, angles], axis=-1)
    boxes_f = jnp.repeat(boxes5, _NUM_CLASSES, axis=1)
    scores_f = scores.reshape(B, _MAX_DET * _NUM_CLASSES)
    labels = jnp.tile(jnp.arange(_NUM_CLASSES), _MAX_DET).astype(jnp.float32)
    labels = jnp.broadcast_to(labels[None, :, None], (B, 600, 1))

    dets = jnp.concatenate(
        [boxes_f, scores_f[:, :, None], labels,
         jnp.zeros((B, 600, 1), jnp.float32)], axis=-1)
    pad = jnp.zeros((B, _NL - 600, 8), jnp.float32)
    pad = pad.at[:, :, 5].set(-2.0)
    dets_c = jnp.concatenate([dets, pad], axis=1)
    dets_t = jnp.transpose(dets_c, (0, 2, 1))

    out = _nms_call(dets_c, dets_t)
    return out[:, :_MAX_DET, :7]
